# baseline (speedup 1.0000x reference)
import functools

import jax
import jax.numpy as jnp
from jax import lax
from jax.experimental import pallas as pl
from jax.experimental.pallas import tpu as pltpu
from jax.experimental.pallas import tpu_sc as plsc

N = 10000
E = 320000
MAX_NB = 32
H = 128
NMOLS_PAD = 512

NC, NS = 2, 16
NW = NC * NS


def _mm_body(relu_in, out_dtype, x_ref, w_ref, o_ref):
    x = x_ref[...]
    if relu_in:
        x = jnp.maximum(x, 0.0)
    o_ref[...] = jnp.dot(
        x.astype(jnp.float32), w_ref[...],
        preferred_element_type=jnp.float32).astype(out_dtype)


def _tc_matmul(x, w, relu_in, out_dtype=jnp.float32, block_rows=2000):
    m, k = x.shape
    _, n = w.shape
    grid = m // block_rows
    return pl.pallas_call(
        functools.partial(_mm_body, relu_in, out_dtype),
        grid=(grid,),
        in_specs=[
            pl.BlockSpec((block_rows, k), lambda i: (i, 0)),
            pl.BlockSpec((k, n), lambda i: (0, 0)),
        ],
        out_specs=pl.BlockSpec((block_rows, n), lambda i: (i, 0)),
        out_shape=jax.ShapeDtypeStruct((m, n), out_dtype),
        compiler_params=pltpu.CompilerParams(
            dimension_semantics=("parallel",)),
    )(x, w)


N_PAD = 10240
GS_ATOMS = N_PAD // NW
GS_BLOCKS = GS_ATOMS // 4


def _bf16_decode(w):
    lo = lax.bitcast_convert_type(lax.shift_left(w, 16), jnp.float32)
    hi = lax.bitcast_convert_type(
        w & jnp.full((16,), -65536, jnp.int32), jnp.float32)
    return lo, hi


def _make_gathersum(apply_relu, packed):
    mesh = plsc.VectorSubcoreMesh(core_axis_name="c", subcore_axis_name="s")
    row_w = H // 2 if packed else H

    @functools.partial(
        pl.kernel,
        out_type=jax.ShapeDtypeStruct((N_PAD, H), jnp.float32),
        mesh=mesh,
        scratch_types=[
            pltpu.VMEM((GS_BLOCKS, 128), jnp.int32),
            pltpu.VMEM((128, row_w),
                       jnp.int32 if packed else jnp.float32),
            pltpu.VMEM((128, row_w),
                       jnp.int32 if packed else jnp.float32),
            pltpu.VMEM((GS_ATOMS, H), jnp.float32),
            pltpu.SemaphoreType.DMA,
            pltpu.SemaphoreType.DMA,
        ],
        compiler_params=pltpu.CompilerParams(use_tc_tiling_on_sc=False),
    )
    def gsum(mw_hbm, a2b_hbm, a_hbm, idx_v, rows0_v, rows1_v, aout_v,
             sem0, sem1):
        wid = lax.axis_index("s") * NC + lax.axis_index("c")
        base = wid * GS_ATOMS
        pltpu.sync_copy(a2b_hbm.at[pl.ds(wid * GS_BLOCKS, GS_BLOCKS)], idx_v)

        def issue(b, rows_v, sem):
            pltpu.async_copy(mw_hbm.at[idx_v.at[b]], rows_v, sem)

        def drain(rows_v, sem):
            pltpu.make_async_copy(mw_hbm.at[pl.ds(0, 128)], rows_v, sem).wait()

        def process_f32(b, rows_v):
            for j in range(4):
                for c in range(H // 16):
                    sl = pl.ds(c * 16, 16)
                    r0 = rows_v[j * MAX_NB, sl]
                    if apply_relu:
                        r0 = jnp.maximum(r0, 0.0)
                    acc = r0
                    for r in range(1, MAX_NB):
                        v = rows_v[j * MAX_NB + r, sl]
                        if apply_relu:
                            v = jnp.maximum(v, 0.0)
                        acc = acc + v
                    aout_v[b * 4 + j, sl] = acc

        def process_packed(b, rows_v):
            for j in range(4):
                for c in range(H // 32):
                    sl = pl.ds(c * 16, 16)
                    lo, hi = _bf16_decode(rows_v[j * MAX_NB, sl])
                    acc_lo, acc_hi = lo, hi
                    for r in range(1, MAX_NB):
                        lo, hi = _bf16_decode(rows_v[j * MAX_NB + r, sl])
                        acc_lo = acc_lo + lo
                        acc_hi = acc_hi + hi
                    aout_v[b * 4 + j, pl.ds(c * 32, 16)] = acc_lo
                    aout_v[b * 4 + j, pl.ds(c * 32 + 16, 16)] = acc_hi

        process = process_packed if packed else process_f32

        issue(0, rows0_v, sem0)

        def pair_body(i, _):
            b0 = 2 * i
            issue(b0 + 1, rows1_v, sem1)
            drain(rows0_v, sem0)
            process(b0, rows0_v)

            @pl.when(i < GS_BLOCKS // 2 - 1)
            def _():
                issue(b0 + 2, rows0_v, sem0)

            drain(rows1_v, sem1)
            process(b0 + 1, rows1_v)
            return 0

        lax.fori_loop(0, GS_BLOCKS // 2, pair_body, 0, unroll=False)
        pltpu.sync_copy(aout_v, a_hbm.at[pl.ds(base, GS_ATOMS)])

    return gsum


_gathersum_bf = _make_gathersum(False, packed=True)
_gathersum_relu = _make_gathersum(True, packed=False)


CB_STRIDE = 10000
CB_EDGES = 10240
CB_EB = 64
CB_BLOCKS = CB_EDGES // CB_EB


def _make_combine():
    mesh = plsc.VectorSubcoreMesh(core_axis_name="c", subcore_axis_name="s")

    @functools.partial(
        pl.kernel,
        out_type=jax.ShapeDtypeStruct((E, H), jnp.float32),
        mesh=mesh,
        scratch_types=[
            pltpu.VMEM((CB_EDGES,), jnp.int32),
            pltpu.VMEM((CB_EDGES,), jnp.int32),
            pltpu.VMEM((CB_EB, H), jnp.float32),
            pltpu.VMEM((CB_EB, H // 2), jnp.int32),
            pltpu.VMEM((CB_EB, H), jnp.float32),
            pltpu.VMEM((CB_EB, H), jnp.float32),
            pltpu.VMEM((CB_EB, H), jnp.float32),
            pltpu.VMEM((CB_EB, H // 2), jnp.int32),
            pltpu.VMEM((CB_EB, H), jnp.float32),
            pltpu.VMEM((CB_EB, H), jnp.float32),
            pltpu.SemaphoreType.DMA, pltpu.SemaphoreType.DMA,
            pltpu.SemaphoreType.DMA, pltpu.SemaphoreType.DMA,
            pltpu.SemaphoreType.DMA, pltpu.SemaphoreType.DMA,
            pltpu.SemaphoreType.DMA, pltpu.SemaphoreType.DMA,
        ],
        compiler_params=pltpu.CompilerParams(use_tc_tiling_on_sc=False),
    )
    def combine(inp_hbm, a_hbm, mw_hbm, b2a_hbm, b2revb_hbm, out_hbm,
                aidx_v, ridx_v,
                ar0, mr0, ir0, or0, ar1, mr1, ir1, or1,
                sa0, sm0, si0, so0, sa1, sm1, si1, so1):
        wid = lax.axis_index("s") * NC + lax.axis_index("c")
        base = jnp.minimum(wid * CB_STRIDE, E - CB_EDGES)
        pltpu.sync_copy(b2a_hbm.at[pl.ds(base, CB_EDGES)], aidx_v)
        pltpu.sync_copy(b2revb_hbm.at[pl.ds(base, CB_EDGES)], ridx_v)

        def issue_in(b, ar, mr, ir, sa, sm, si):
            off = b * CB_EB
            pltpu.async_copy(a_hbm.at[aidx_v.at[pl.ds(off, CB_EB)]], ar, sa)
            pltpu.async_copy(mw_hbm.at[ridx_v.at[pl.ds(off, CB_EB)]], mr, sm)
            pltpu.async_copy(inp_hbm.at[pl.ds(base + off, CB_EB)], ir, si)

        def drain_in(ar, mr, ir, sa, sm, si):
            pltpu.make_async_copy(a_hbm.at[pl.ds(0, CB_EB)], ar, sa).wait()
            pltpu.make_async_copy(mw_hbm.at[pl.ds(0, CB_EB)], mr, sm).wait()
            pltpu.make_async_copy(inp_hbm.at[pl.ds(0, CB_EB)], ir, si).wait()

        def drain_out(orv, so):
            pltpu.make_async_copy(orv, out_hbm.at[pl.ds(0, CB_EB)], so).wait()

        def compute(ar, mr, ir, orv):
            def row_body(r, _):
                for c in range(H // 32):
                    mlo, mhi = _bf16_decode(mr[r, pl.ds(c * 16, 16)])
                    sl0 = pl.ds(c * 32, 16)
                    sl1 = pl.ds(c * 32 + 16, 16)
                    orv[r, sl0] = ir[r, sl0] + ar[r, sl0] - mlo
                    orv[r, sl1] = ir[r, sl1] + ar[r, sl1] - mhi
                return 0
            lax.fori_loop(0, CB_EB, row_body, 0, unroll=4)

        issue_in(0, ar0, mr0, ir0, sa0, sm0, si0)

        def pair_body(i, _):
            b0 = 2 * i
            issue_in(b0 + 1, ar1, mr1, ir1, sa1, sm1, si1)
            drain_in(ar0, mr0, ir0, sa0, sm0, si0)

            @pl.when(i > 0)
            def _():
                drain_out(or0, so0)

            compute(ar0, mr0, ir0, or0)
            pltpu.async_copy(or0, out_hbm.at[pl.ds(base + b0 * CB_EB, CB_EB)],
                             so0)

            @pl.when(i < CB_BLOCKS // 2 - 1)
            def _():
                issue_in(b0 + 2, ar0, mr0, ir0, sa0, sm0, si0)

            drain_in(ar1, mr1, ir1, sa1, sm1, si1)

            @pl.when(i > 0)
            def _():
                drain_out(or1, so1)

            compute(ar1, mr1, ir1, or1)
            pltpu.async_copy(
                or1, out_hbm.at[pl.ds(base + (b0 + 1) * CB_EB, CB_EB)], so1)
            return 0

        lax.fori_loop(0, CB_BLOCKS // 2, pair_body, 0, unroll=False)
        drain_out(or0, so0)
        drain_out(or1, so1)

    return combine


_combine = _make_combine()


RO_BLOCK = 2000
RO_GRID = N // RO_BLOCK


def _ro_body(fa_ref, a3_ref, wo1_ref, wo2_ref, bo_ref, mid_ref, out_ref,
             msum_ref, cnt_ref):
    i = pl.program_id(0)

    @pl.when(i == 0)
    def _():
        msum_ref[...] = jnp.zeros_like(msum_ref)
        cnt_ref[...] = jnp.zeros_like(cnt_ref)

    h = jnp.dot(fa_ref[...], wo1_ref[...], preferred_element_type=jnp.float32)
    h = h + jnp.dot(a3_ref[...], wo2_ref[...],
                    preferred_element_type=jnp.float32)
    h = jnp.maximum(h + bo_ref[...], 0.0)

    mid_row = mid_ref[0]
    mask_t = (lax.broadcasted_iota(jnp.int32, (NMOLS_PAD, RO_BLOCK), 0)
              == mid_row).astype(jnp.float32)
    msum_ref[...] += jnp.dot(mask_t, h, preferred_element_type=jnp.float32)
    cnt_ref[...] += jnp.broadcast_to(
        jnp.sum(mask_t, axis=1, keepdims=True), (NMOLS_PAD, H))

    @pl.when(i == RO_GRID - 1)
    def _():
        out_ref[...] = msum_ref[...] / jnp.maximum(cnt_ref[...], 1.0)


def _tc_readout(f_atoms, a3, wo1, wo2, bo, mol_ids_2d):
    return pl.pallas_call(
        _ro_body,
        grid=(RO_GRID,),
        in_specs=[
            pl.BlockSpec((RO_BLOCK, H), lambda i: (i, 0)),
            pl.BlockSpec((RO_BLOCK, H), lambda i: (i, 0)),
            pl.BlockSpec((H, H), lambda i: (0, 0)),
            pl.BlockSpec((H, H), lambda i: (0, 0)),
            pl.BlockSpec((1, H), lambda i: (0, 0)),
            pl.BlockSpec((1, 1, RO_BLOCK), lambda i: (i, 0, 0)),
        ],
        out_specs=pl.BlockSpec((NMOLS_PAD, H), lambda i: (0, 0)),
        out_shape=jax.ShapeDtypeStruct((NMOLS_PAD, H), jnp.float32),
        scratch_shapes=[
            pltpu.VMEM((NMOLS_PAD, H), jnp.float32),
            pltpu.VMEM((NMOLS_PAD, H), jnp.float32),
        ],
        compiler_params=pltpu.CompilerParams(
            dimension_semantics=("arbitrary",)),
    )(f_atoms, a3, wo1, wo2, bo, mol_ids_2d)


def _interleave_perm():
    perm = []
    for g in range(H // 32):
        for i in range(16):
            perm.append(32 * g + i)
            perm.append(32 * g + 16 + i)
    return perm


_MW_PERM = tuple(_interleave_perm())


def kernel(f_atoms, f_bonds, a2b, b2a, b2revb, mol_ids, W_i, W_h, W_o, b_o):
    a2b = a2b.astype(jnp.int32)
    b2a = b2a.astype(jnp.int32)
    b2revb = b2revb.astype(jnp.int32)
    a2b_pad = jnp.zeros((N_PAD, MAX_NB), jnp.int32).at[:N].set(a2b)
    a2b_r = a2b_pad.reshape(N_PAD * MAX_NB // 128, 128)

    inp = _tc_matmul(f_bonds, W_i, relu_in=False)
    w_h_perm = W_h[:, jnp.array(_MW_PERM, jnp.int32)]

    pre = inp
    for _ in range(2):
        mw = _tc_matmul(pre, w_h_perm, relu_in=True,
                        out_dtype=jnp.bfloat16)
        mw_i = lax.bitcast_convert_type(
            mw.reshape(E, H // 2, 2), jnp.int32)
        a_sum = _gathersum_bf(mw_i, a2b_r)
        pre = _combine(inp, a_sum, mw_i, b2a, b2revb)

    a3 = _gathersum_relu(pre, a2b_r)[:N]

    mol_ids_3d = mol_ids.astype(jnp.int32).reshape(RO_GRID, 1, RO_BLOCK)
    wo1 = W_o[:H]
    wo2 = W_o[H:]
    bo = b_o.reshape(1, H)
    mol_vecs = _tc_readout(f_atoms, a3, wo1, wo2, bo, mol_ids_3d)
    return mol_vecs[:500]


# speedup vs baseline: 1.6725x; 1.6725x over previous
import functools

import jax
import jax.numpy as jnp
from jax import lax
from jax.experimental import pallas as pl
from jax.experimental.pallas import tpu as pltpu
from jax.experimental.pallas import tpu_sc as plsc

N = 10000
E = 320000
MAX_NB = 32
H = 128
NMOLS_PAD = 512

NC, NS = 2, 16
NW = NC * NS


def _rne_bf16_bits(b):
    return b + 0x7FFF + (lax.shift_right_arithmetic(b, 16) & 1)


def _mm_body(relu_in, pack_bf16, x_ref, w_ref, o_ref):
    x = x_ref[...]
    if relu_in:
        x = jnp.maximum(x, 0.0)
    y = jnp.dot(x.astype(jnp.float32), w_ref[...],
                preferred_element_type=jnp.float32)
    if pack_bf16:
        n = y.shape[1]
        b = lax.bitcast_convert_type(y, jnp.int32)
        lo = _rne_bf16_bits(b[:, : n // 2])
        hi = _rne_bf16_bits(b[:, n // 2:])
        o_ref[...] = ((lax.shift_right_arithmetic(lo, 16) & 0xFFFF)
                      | (hi & jnp.int32(-65536)))
    else:
        o_ref[...] = y


def _tc_matmul(x, w, relu_in, pack_bf16=False, block_rows=2000):
    m, k = x.shape
    _, n = w.shape
    n_out = n // 2 if pack_bf16 else n
    out_dtype = jnp.int32 if pack_bf16 else jnp.float32
    grid = m // block_rows
    return pl.pallas_call(
        functools.partial(_mm_body, relu_in, pack_bf16),
        grid=(grid,),
        in_specs=[
            pl.BlockSpec((block_rows, k), lambda i: (i, 0)),
            pl.BlockSpec((k, n), lambda i: (0, 0)),
        ],
        out_specs=pl.BlockSpec((block_rows, n_out), lambda i: (i, 0)),
        out_shape=jax.ShapeDtypeStruct((m, n_out), out_dtype),
        compiler_params=pltpu.CompilerParams(
            dimension_semantics=("parallel",)),
    )(x, w)


N_PAD = 10240
GS_ATOMS = N_PAD // NW
GS_BLOCKS = GS_ATOMS // 4


def _bf16_decode(w):
    lo = lax.bitcast_convert_type(lax.shift_left(w, 16), jnp.float32)
    hi = lax.bitcast_convert_type(
        w & jnp.full((16,), -65536, jnp.int32), jnp.float32)
    return lo, hi


def _make_gathersum(apply_relu, packed):
    mesh = plsc.VectorSubcoreMesh(core_axis_name="c", subcore_axis_name="s")
    row_w = H // 2 if packed else H

    @functools.partial(
        pl.kernel,
        out_type=jax.ShapeDtypeStruct((N_PAD, H), jnp.float32),
        mesh=mesh,
        scratch_types=[
            pltpu.VMEM((GS_BLOCKS, 128), jnp.int32),
            pltpu.VMEM((128, row_w),
                       jnp.int32 if packed else jnp.float32),
            pltpu.VMEM((128, row_w),
                       jnp.int32 if packed else jnp.float32),
            pltpu.VMEM((GS_ATOMS, H), jnp.float32),
            pltpu.SemaphoreType.DMA,
            pltpu.SemaphoreType.DMA,
        ],
        compiler_params=pltpu.CompilerParams(use_tc_tiling_on_sc=False),
    )
    def gsum(mw_hbm, a2b_hbm, a_hbm, idx_v, rows0_v, rows1_v, aout_v,
             sem0, sem1):
        wid = lax.axis_index("s") * NC + lax.axis_index("c")
        base = wid * GS_ATOMS
        pltpu.sync_copy(a2b_hbm.at[pl.ds(wid * GS_BLOCKS, GS_BLOCKS)], idx_v)

        def issue(b, rows_v, sem):
            pltpu.async_copy(mw_hbm.at[idx_v.at[b]], rows_v, sem)

        def drain(rows_v, sem):
            pltpu.make_async_copy(mw_hbm.at[pl.ds(0, 128)], rows_v, sem).wait()

        def process_f32(b, rows_v):
            for j in range(4):
                for c in range(H // 16):
                    sl = pl.ds(c * 16, 16)
                    r0 = rows_v[j * MAX_NB, sl]
                    if apply_relu:
                        r0 = jnp.maximum(r0, 0.0)
                    acc = r0
                    for r in range(1, MAX_NB):
                        v = rows_v[j * MAX_NB + r, sl]
                        if apply_relu:
                            v = jnp.maximum(v, 0.0)
                        acc = acc + v
                    aout_v[b * 4 + j, sl] = acc

        def process_packed(b, rows_v):
            for j in range(4):
                for c in range(H // 32):
                    sl = pl.ds(c * 16, 16)
                    lo, hi = _bf16_decode(rows_v[j * MAX_NB, sl])
                    acc_lo, acc_hi = lo, hi
                    for r in range(1, MAX_NB):
                        lo, hi = _bf16_decode(rows_v[j * MAX_NB + r, sl])
                        acc_lo = acc_lo + lo
                        acc_hi = acc_hi + hi
                    aout_v[b * 4 + j, pl.ds(c * 16, 16)] = acc_lo
                    aout_v[b * 4 + j, pl.ds(H // 2 + c * 16, 16)] = acc_hi

        process = process_packed if packed else process_f32

        issue(0, rows0_v, sem0)

        def pair_body(i, _):
            b0 = 2 * i
            issue(b0 + 1, rows1_v, sem1)
            drain(rows0_v, sem0)
            process(b0, rows0_v)

            @pl.when(i < GS_BLOCKS // 2 - 1)
            def _():
                issue(b0 + 2, rows0_v, sem0)

            drain(rows1_v, sem1)
            process(b0 + 1, rows1_v)
            return 0

        lax.fori_loop(0, GS_BLOCKS // 2, pair_body, 0, unroll=False)
        pltpu.sync_copy(aout_v, a_hbm.at[pl.ds(base, GS_ATOMS)])

    return gsum


_gathersum_bf = _make_gathersum(False, packed=True)
_gathersum_relu = _make_gathersum(True, packed=False)


CB_STRIDE = 10000
CB_EDGES = 10240
CB_EB = 64
CB_BLOCKS = CB_EDGES // CB_EB


def _make_combine():
    mesh = plsc.VectorSubcoreMesh(core_axis_name="c", subcore_axis_name="s")

    @functools.partial(
        pl.kernel,
        out_type=jax.ShapeDtypeStruct((E, H), jnp.float32),
        mesh=mesh,
        scratch_types=[
            pltpu.VMEM((CB_EDGES,), jnp.int32),
            pltpu.VMEM((CB_EDGES,), jnp.int32),
            pltpu.VMEM((CB_EB, H), jnp.float32),
            pltpu.VMEM((CB_EB, H // 2), jnp.int32),
            pltpu.VMEM((CB_EB, H), jnp.float32),
            pltpu.VMEM((CB_EB, H), jnp.float32),
            pltpu.VMEM((CB_EB, H), jnp.float32),
            pltpu.VMEM((CB_EB, H // 2), jnp.int32),
            pltpu.VMEM((CB_EB, H), jnp.float32),
            pltpu.VMEM((CB_EB, H), jnp.float32),
            pltpu.SemaphoreType.DMA, pltpu.SemaphoreType.DMA,
            pltpu.SemaphoreType.DMA, pltpu.SemaphoreType.DMA,
            pltpu.SemaphoreType.DMA, pltpu.SemaphoreType.DMA,
            pltpu.SemaphoreType.DMA, pltpu.SemaphoreType.DMA,
        ],
        compiler_params=pltpu.CompilerParams(use_tc_tiling_on_sc=False),
    )
    def combine(inp_hbm, a_hbm, mw_hbm, b2a_hbm, b2revb_hbm, out_hbm,
                aidx_v, ridx_v,
                ar0, mr0, ir0, or0, ar1, mr1, ir1, or1,
                sa0, sm0, si0, so0, sa1, sm1, si1, so1):
        wid = lax.axis_index("s") * NC + lax.axis_index("c")
        base = jnp.minimum(wid * CB_STRIDE, E - CB_EDGES)
        pltpu.sync_copy(b2a_hbm.at[pl.ds(base, CB_EDGES)], aidx_v)
        pltpu.sync_copy(b2revb_hbm.at[pl.ds(base, CB_EDGES)], ridx_v)

        def issue_in(b, ar, mr, ir, sa, sm, si):
            off = b * CB_EB
            pltpu.async_copy(a_hbm.at[aidx_v.at[pl.ds(off, CB_EB)]], ar, sa)
            pltpu.async_copy(mw_hbm.at[ridx_v.at[pl.ds(off, CB_EB)]], mr, sm)
            pltpu.async_copy(inp_hbm.at[pl.ds(base + off, CB_EB)], ir, si)

        def drain_in(ar, mr, ir, sa, sm, si):
            pltpu.make_async_copy(a_hbm.at[pl.ds(0, CB_EB)], ar, sa).wait()
            pltpu.make_async_copy(mw_hbm.at[pl.ds(0, CB_EB)], mr, sm).wait()
            pltpu.make_async_copy(inp_hbm.at[pl.ds(0, CB_EB)], ir, si).wait()

        def drain_out(orv, so):
            pltpu.make_async_copy(orv, out_hbm.at[pl.ds(0, CB_EB)], so).wait()

        def compute(ar, mr, ir, orv):
            def row_body(r, _):
                for c in range(H // 32):
                    mlo, mhi = _bf16_decode(mr[r, pl.ds(c * 16, 16)])
                    sl0 = pl.ds(c * 16, 16)
                    sl1 = pl.ds(H // 2 + c * 16, 16)
                    orv[r, sl0] = ir[r, sl0] + ar[r, sl0] - mlo
                    orv[r, sl1] = ir[r, sl1] + ar[r, sl1] - mhi
                return 0
            lax.fori_loop(0, CB_EB, row_body, 0, unroll=4)

        issue_in(0, ar0, mr0, ir0, sa0, sm0, si0)

        def pair_body(i, _):
            b0 = 2 * i
            issue_in(b0 + 1, ar1, mr1, ir1, sa1, sm1, si1)
            drain_in(ar0, mr0, ir0, sa0, sm0, si0)

            @pl.when(i > 0)
            def _():
                drain_out(or0, so0)

            compute(ar0, mr0, ir0, or0)
            pltpu.async_copy(or0, out_hbm.at[pl.ds(base + b0 * CB_EB, CB_EB)],
                             so0)

            @pl.when(i < CB_BLOCKS // 2 - 1)
            def _():
                issue_in(b0 + 2, ar0, mr0, ir0, sa0, sm0, si0)

            drain_in(ar1, mr1, ir1, sa1, sm1, si1)

            @pl.when(i > 0)
            def _():
                drain_out(or1, so1)

            compute(ar1, mr1, ir1, or1)
            pltpu.async_copy(
                or1, out_hbm.at[pl.ds(base + (b0 + 1) * CB_EB, CB_EB)], so1)
            return 0

        lax.fori_loop(0, CB_BLOCKS // 2, pair_body, 0, unroll=False)
        drain_out(or0, so0)
        drain_out(or1, so1)

    return combine


_combine = _make_combine()


RO_BLOCK = 2000
RO_GRID = N // RO_BLOCK


def _ro_body(fa_ref, a3_ref, wo1_ref, wo2_ref, bo_ref, mid_ref, out_ref,
             msum_ref, cnt_ref):
    i = pl.program_id(0)

    @pl.when(i == 0)
    def _():
        msum_ref[...] = jnp.zeros_like(msum_ref)
        cnt_ref[...] = jnp.zeros_like(cnt_ref)

    h = jnp.dot(fa_ref[...], wo1_ref[...], preferred_element_type=jnp.float32)
    h = h + jnp.dot(a3_ref[...], wo2_ref[...],
                    preferred_element_type=jnp.float32)
    h = jnp.maximum(h + bo_ref[...], 0.0)

    mid_row = mid_ref[0]
    mask_t = (lax.broadcasted_iota(jnp.int32, (NMOLS_PAD, RO_BLOCK), 0)
              == mid_row).astype(jnp.float32)
    msum_ref[...] += jnp.dot(mask_t, h, preferred_element_type=jnp.float32)
    cnt_ref[...] += jnp.broadcast_to(
        jnp.sum(mask_t, axis=1, keepdims=True), (NMOLS_PAD, H))

    @pl.when(i == RO_GRID - 1)
    def _():
        out_ref[...] = msum_ref[...] / jnp.maximum(cnt_ref[...], 1.0)


def _tc_readout(f_atoms, a3, wo1, wo2, bo, mol_ids_2d):
    return pl.pallas_call(
        _ro_body,
        grid=(RO_GRID,),
        in_specs=[
            pl.BlockSpec((RO_BLOCK, H), lambda i: (i, 0)),
            pl.BlockSpec((RO_BLOCK, H), lambda i: (i, 0)),
            pl.BlockSpec((H, H), lambda i: (0, 0)),
            pl.BlockSpec((H, H), lambda i: (0, 0)),
            pl.BlockSpec((1, H), lambda i: (0, 0)),
            pl.BlockSpec((1, 1, RO_BLOCK), lambda i: (i, 0, 0)),
        ],
        out_specs=pl.BlockSpec((NMOLS_PAD, H), lambda i: (0, 0)),
        out_shape=jax.ShapeDtypeStruct((NMOLS_PAD, H), jnp.float32),
        scratch_shapes=[
            pltpu.VMEM((NMOLS_PAD, H), jnp.float32),
            pltpu.VMEM((NMOLS_PAD, H), jnp.float32),
        ],
        compiler_params=pltpu.CompilerParams(
            dimension_semantics=("arbitrary",)),
    )(f_atoms, a3, wo1, wo2, bo, mol_ids_2d)


def kernel(f_atoms, f_bonds, a2b, b2a, b2revb, mol_ids, W_i, W_h, W_o, b_o):
    a2b = a2b.astype(jnp.int32)
    b2a = b2a.astype(jnp.int32)
    b2revb = b2revb.astype(jnp.int32)
    a2b_pad = jnp.zeros((N_PAD, MAX_NB), jnp.int32).at[:N].set(a2b)
    a2b_r = a2b_pad.reshape(N_PAD * MAX_NB // 128, 128)

    inp = _tc_matmul(f_bonds, W_i, relu_in=False)

    pre = inp
    for _ in range(2):
        mw_i = _tc_matmul(pre, W_h, relu_in=True,
                          pack_bf16=True)
        a_sum = _gathersum_bf(mw_i, a2b_r)
        pre = _combine(inp, a_sum, mw_i, b2a, b2revb)

    a3 = _gathersum_relu(pre, a2b_r)[:N]

    mol_ids_3d = mol_ids.astype(jnp.int32).reshape(RO_GRID, 1, RO_BLOCK)
    wo1 = W_o[:H]
    wo2 = W_o[H:]
    bo = b_o.reshape(1, H)
    mol_vecs = _tc_readout(f_atoms, a3, wo1, wo2, bo, mol_ids_3d)
    return mol_vecs[:500]


# speedup vs baseline: 1.7054x; 1.0197x over previous
import functools

import jax
import jax.numpy as jnp
from jax import lax
from jax.experimental import pallas as pl
from jax.experimental.pallas import tpu as pltpu
from jax.experimental.pallas import tpu_sc as plsc

N = 10000
E = 320000
MAX_NB = 32
H = 128
NMOLS_PAD = 512

NC, NS = 2, 16
NW = NC * NS


def _rne_bf16_bits(b):
    return b + 0x7FFF + (lax.shift_right_arithmetic(b, 16) & 1)


def _mm_body(relu_in, pack_bf16, unpack_in, x_ref, w_ref, o_ref):
    w = w_ref[...]
    if unpack_in:
        xi = x_ref[...]
        lo = lax.bitcast_convert_type(lax.shift_left(xi, 16), jnp.float32)
        hi = lax.bitcast_convert_type(xi & jnp.int32(-65536), jnp.float32)
        if relu_in:
            lo = jnp.maximum(lo, 0.0)
            hi = jnp.maximum(hi, 0.0)
        k2 = w.shape[0] // 2
        y = (jnp.dot(lo, w[:k2], preferred_element_type=jnp.float32)
             + jnp.dot(hi, w[k2:], preferred_element_type=jnp.float32))
    else:
        x = x_ref[...]
        if relu_in:
            x = jnp.maximum(x, 0.0)
        y = jnp.dot(x, w, preferred_element_type=jnp.float32)
    if pack_bf16:
        n = y.shape[1]
        b = lax.bitcast_convert_type(y, jnp.int32)
        lo = _rne_bf16_bits(b[:, : n // 2])
        hi = _rne_bf16_bits(b[:, n // 2:])
        o_ref[...] = ((lax.shift_right_arithmetic(lo, 16) & 0xFFFF)
                      | (hi & jnp.int32(-65536)))
    else:
        o_ref[...] = y


def _tc_matmul(x, w, relu_in, pack_bf16=False, unpack_in=False,
               block_rows=2000):
    m, k_in = x.shape
    k, n = w.shape
    n_out = n // 2 if pack_bf16 else n
    out_dtype = jnp.int32 if pack_bf16 else jnp.float32
    grid = m // block_rows
    return pl.pallas_call(
        functools.partial(_mm_body, relu_in, pack_bf16, unpack_in),
        grid=(grid,),
        in_specs=[
            pl.BlockSpec((block_rows, k_in), lambda i: (i, 0)),
            pl.BlockSpec((k, n), lambda i: (0, 0)),
        ],
        out_specs=pl.BlockSpec((block_rows, n_out), lambda i: (i, 0)),
        out_shape=jax.ShapeDtypeStruct((m, n_out), out_dtype),
        compiler_params=pltpu.CompilerParams(
            dimension_semantics=("parallel",)),
    )(x, w)


N_PAD = 10240
GS_ATOMS = N_PAD // NW
GS_BLOCKS = GS_ATOMS // 4


def _bf16_decode(w):
    lo = lax.bitcast_convert_type(lax.shift_left(w, 16), jnp.float32)
    hi = lax.bitcast_convert_type(
        w & jnp.full((16,), -65536, jnp.int32), jnp.float32)
    return lo, hi


def _make_gathersum(apply_relu, packed):
    mesh = plsc.VectorSubcoreMesh(core_axis_name="c", subcore_axis_name="s")
    row_w = H // 2 if packed else H

    @functools.partial(
        pl.kernel,
        out_type=jax.ShapeDtypeStruct((N_PAD, H), jnp.float32),
        mesh=mesh,
        scratch_types=[
            pltpu.VMEM((GS_BLOCKS, 128), jnp.int32),
            pltpu.VMEM((128, row_w),
                       jnp.int32 if packed else jnp.float32),
            pltpu.VMEM((128, row_w),
                       jnp.int32 if packed else jnp.float32),
            pltpu.VMEM((GS_ATOMS, H), jnp.float32),
            pltpu.SemaphoreType.DMA,
            pltpu.SemaphoreType.DMA,
        ],
        compiler_params=pltpu.CompilerParams(use_tc_tiling_on_sc=False),
    )
    def gsum(mw_hbm, a2b_hbm, a_hbm, idx_v, rows0_v, rows1_v, aout_v,
             sem0, sem1):
        wid = lax.axis_index("s") * NC + lax.axis_index("c")
        base = wid * GS_ATOMS
        pltpu.sync_copy(a2b_hbm.at[pl.ds(wid * GS_BLOCKS, GS_BLOCKS)], idx_v)

        def issue(b, rows_v, sem):
            pltpu.async_copy(mw_hbm.at[idx_v.at[b]], rows_v, sem)

        def drain(rows_v, sem):
            pltpu.make_async_copy(mw_hbm.at[pl.ds(0, 128)], rows_v, sem).wait()

        def process_f32(b, rows_v):
            for j in range(4):
                for c in range(H // 16):
                    sl = pl.ds(c * 16, 16)
                    r0 = rows_v[j * MAX_NB, sl]
                    if apply_relu:
                        r0 = jnp.maximum(r0, 0.0)
                    acc = r0
                    for r in range(1, MAX_NB):
                        v = rows_v[j * MAX_NB + r, sl]
                        if apply_relu:
                            v = jnp.maximum(v, 0.0)
                        acc = acc + v
                    aout_v[b * 4 + j, sl] = acc

        def process_packed(b, rows_v):
            for j in range(4):
                for c in range(H // 32):
                    sl = pl.ds(c * 16, 16)
                    lo, hi = _bf16_decode(rows_v[j * MAX_NB, sl])
                    if apply_relu:
                        lo = jnp.maximum(lo, 0.0)
                        hi = jnp.maximum(hi, 0.0)
                    acc_lo, acc_hi = lo, hi
                    for r in range(1, MAX_NB):
                        lo, hi = _bf16_decode(rows_v[j * MAX_NB + r, sl])
                        if apply_relu:
                            lo = jnp.maximum(lo, 0.0)
                            hi = jnp.maximum(hi, 0.0)
                        acc_lo = acc_lo + lo
                        acc_hi = acc_hi + hi
                    aout_v[b * 4 + j, pl.ds(c * 16, 16)] = acc_lo
                    aout_v[b * 4 + j, pl.ds(H // 2 + c * 16, 16)] = acc_hi

        process = process_packed if packed else process_f32

        issue(0, rows0_v, sem0)

        def pair_body(i, _):
            b0 = 2 * i
            issue(b0 + 1, rows1_v, sem1)
            drain(rows0_v, sem0)
            process(b0, rows0_v)

            @pl.when(i < GS_BLOCKS // 2 - 1)
            def _():
                issue(b0 + 2, rows0_v, sem0)

            drain(rows1_v, sem1)
            process(b0 + 1, rows1_v)
            return 0

        lax.fori_loop(0, GS_BLOCKS // 2, pair_body, 0, unroll=False)
        pltpu.sync_copy(aout_v, a_hbm.at[pl.ds(base, GS_ATOMS)])

    return gsum


_gathersum_bf = _make_gathersum(False, packed=True)
_gathersum_relu_bf = _make_gathersum(True, packed=True)


CB_STRIDE = 10000
CB_EDGES = 10240
CB_EB = 64
CB_BLOCKS = CB_EDGES // CB_EB


def _make_combine():
    mesh = plsc.VectorSubcoreMesh(core_axis_name="c", subcore_axis_name="s")

    @functools.partial(
        pl.kernel,
        out_type=jax.ShapeDtypeStruct((E, H // 2), jnp.int32),
        mesh=mesh,
        scratch_types=[
            pltpu.VMEM((CB_EDGES,), jnp.int32),
            pltpu.VMEM((CB_EDGES,), jnp.int32),
            pltpu.VMEM((CB_EB, H), jnp.float32),
            pltpu.VMEM((CB_EB, H // 2), jnp.int32),
            pltpu.VMEM((CB_EB, H), jnp.float32),
            pltpu.VMEM((CB_EB, H // 2), jnp.int32),
            pltpu.VMEM((CB_EB, H), jnp.float32),
            pltpu.VMEM((CB_EB, H // 2), jnp.int32),
            pltpu.VMEM((CB_EB, H), jnp.float32),
            pltpu.VMEM((CB_EB, H // 2), jnp.int32),
            pltpu.SemaphoreType.DMA, pltpu.SemaphoreType.DMA,
            pltpu.SemaphoreType.DMA, pltpu.SemaphoreType.DMA,
            pltpu.SemaphoreType.DMA, pltpu.SemaphoreType.DMA,
            pltpu.SemaphoreType.DMA, pltpu.SemaphoreType.DMA,
        ],
        compiler_params=pltpu.CompilerParams(use_tc_tiling_on_sc=False),
    )
    def combine(inp_hbm, a_hbm, mw_hbm, b2a_hbm, b2revb_hbm, out_hbm,
                aidx_v, ridx_v,
                ar0, mr0, ir0, or0, ar1, mr1, ir1, or1,
                sa0, sm0, si0, so0, sa1, sm1, si1, so1):
        wid = lax.axis_index("s") * NC + lax.axis_index("c")
        base = jnp.minimum(wid * CB_STRIDE, E - CB_EDGES)
        pltpu.sync_copy(b2a_hbm.at[pl.ds(base, CB_EDGES)], aidx_v)
        pltpu.sync_copy(b2revb_hbm.at[pl.ds(base, CB_EDGES)], ridx_v)

        def issue_in(b, ar, mr, ir, sa, sm, si):
            off = b * CB_EB
            pltpu.async_copy(a_hbm.at[aidx_v.at[pl.ds(off, CB_EB)]], ar, sa)
            pltpu.async_copy(mw_hbm.at[ridx_v.at[pl.ds(off, CB_EB)]], mr, sm)
            pltpu.async_copy(inp_hbm.at[pl.ds(base + off, CB_EB)], ir, si)

        def drain_in(ar, mr, ir, sa, sm, si):
            pltpu.make_async_copy(a_hbm.at[pl.ds(0, CB_EB)], ar, sa).wait()
            pltpu.make_async_copy(mw_hbm.at[pl.ds(0, CB_EB)], mr, sm).wait()
            pltpu.make_async_copy(inp_hbm.at[pl.ds(0, CB_EB)], ir, si).wait()

        def drain_out(orv, so):
            pltpu.make_async_copy(orv, out_hbm.at[pl.ds(0, CB_EB)], so).wait()

        def compute(ar, mr, ir, orv):
            def row_body(r, _):
                for c in range(H // 32):
                    mlo, mhi = _bf16_decode(mr[r, pl.ds(c * 16, 16)])
                    sl0 = pl.ds(c * 16, 16)
                    sl1 = pl.ds(H // 2 + c * 16, 16)
                    olo = ir[r, sl0] + ar[r, sl0] - mlo
                    ohi = ir[r, sl1] + ar[r, sl1] - mhi
                    blo = _rne_bf16_bits(lax.bitcast_convert_type(
                        olo, jnp.int32))
                    bhi = _rne_bf16_bits(lax.bitcast_convert_type(
                        ohi, jnp.int32))
                    orv[r, pl.ds(c * 16, 16)] = (
                        (lax.shift_right_arithmetic(blo, 16) & 0xFFFF)
                        | (bhi & jnp.int32(-65536)))
                return 0
            lax.fori_loop(0, CB_EB, row_body, 0, unroll=4)

        issue_in(0, ar0, mr0, ir0, sa0, sm0, si0)

        def pair_body(i, _):
            b0 = 2 * i
            issue_in(b0 + 1, ar1, mr1, ir1, sa1, sm1, si1)
            drain_in(ar0, mr0, ir0, sa0, sm0, si0)

            @pl.when(i > 0)
            def _():
                drain_out(or0, so0)

            compute(ar0, mr0, ir0, or0)
            pltpu.async_copy(or0, out_hbm.at[pl.ds(base + b0 * CB_EB, CB_EB)],
                             so0)

            @pl.when(i < CB_BLOCKS // 2 - 1)
            def _():
                issue_in(b0 + 2, ar0, mr0, ir0, sa0, sm0, si0)

            drain_in(ar1, mr1, ir1, sa1, sm1, si1)

            @pl.when(i > 0)
            def _():
                drain_out(or1, so1)

            compute(ar1, mr1, ir1, or1)
            pltpu.async_copy(
                or1, out_hbm.at[pl.ds(base + (b0 + 1) * CB_EB, CB_EB)], so1)
            return 0

        lax.fori_loop(0, CB_BLOCKS // 2, pair_body, 0, unroll=False)
        drain_out(or0, so0)
        drain_out(or1, so1)

    return combine


_combine = _make_combine()


RO_BLOCK = 2000
RO_GRID = N // RO_BLOCK


def _ro_body(fa_ref, a3_ref, wo1_ref, wo2_ref, bo_ref, mid_ref, out_ref,
             msum_ref, cnt_ref):
    i = pl.program_id(0)

    @pl.when(i == 0)
    def _():
        msum_ref[...] = jnp.zeros_like(msum_ref)
        cnt_ref[...] = jnp.zeros_like(cnt_ref)

    h = jnp.dot(fa_ref[...], wo1_ref[...], preferred_element_type=jnp.float32)
    h = h + jnp.dot(a3_ref[...], wo2_ref[...],
                    preferred_element_type=jnp.float32)
    h = jnp.maximum(h + bo_ref[...], 0.0)

    mid_row = mid_ref[0]
    mask_t = (lax.broadcasted_iota(jnp.int32, (NMOLS_PAD, RO_BLOCK), 0)
              == mid_row).astype(jnp.float32)
    msum_ref[...] += jnp.dot(mask_t, h, preferred_element_type=jnp.float32)
    cnt_ref[...] += jnp.broadcast_to(
        jnp.sum(mask_t, axis=1, keepdims=True), (NMOLS_PAD, H))

    @pl.when(i == RO_GRID - 1)
    def _():
        out_ref[...] = msum_ref[...] / jnp.maximum(cnt_ref[...], 1.0)


def _tc_readout(f_atoms, a3, wo1, wo2, bo, mol_ids_2d):
    return pl.pallas_call(
        _ro_body,
        grid=(RO_GRID,),
        in_specs=[
            pl.BlockSpec((RO_BLOCK, H), lambda i: (i, 0)),
            pl.BlockSpec((RO_BLOCK, H), lambda i: (i, 0)),
            pl.BlockSpec((H, H), lambda i: (0, 0)),
            pl.BlockSpec((H, H), lambda i: (0, 0)),
            pl.BlockSpec((1, H), lambda i: (0, 0)),
            pl.BlockSpec((1, 1, RO_BLOCK), lambda i: (i, 0, 0)),
        ],
        out_specs=pl.BlockSpec((NMOLS_PAD, H), lambda i: (0, 0)),
        out_shape=jax.ShapeDtypeStruct((NMOLS_PAD, H), jnp.float32),
        scratch_shapes=[
            pltpu.VMEM((NMOLS_PAD, H), jnp.float32),
            pltpu.VMEM((NMOLS_PAD, H), jnp.float32),
        ],
        compiler_params=pltpu.CompilerParams(
            dimension_semantics=("arbitrary",)),
    )(f_atoms, a3, wo1, wo2, bo, mol_ids_2d)


def kernel(f_atoms, f_bonds, a2b, b2a, b2revb, mol_ids, W_i, W_h, W_o, b_o):
    a2b = a2b.astype(jnp.int32)
    b2a = b2a.astype(jnp.int32)
    b2revb = b2revb.astype(jnp.int32)
    a2b_pad = jnp.zeros((N_PAD, MAX_NB), jnp.int32).at[:N].set(a2b)
    a2b_r = a2b_pad.reshape(N_PAD * MAX_NB // 128, 128)

    inp = _tc_matmul(f_bonds, W_i, relu_in=False)

    pre = inp
    for it in range(2):
        mw_i = _tc_matmul(pre, W_h, relu_in=True,
                          pack_bf16=True,
                          unpack_in=(it > 0))
        a_sum = _gathersum_bf(mw_i, a2b_r)
        pre = _combine(inp, a_sum, mw_i, b2a, b2revb)

    a3 = _gathersum_relu_bf(pre, a2b_r)[:N]

    mol_ids_3d = mol_ids.astype(jnp.int32).reshape(RO_GRID, 1, RO_BLOCK)
    wo1 = W_o[:H]
    wo2 = W_o[H:]
    bo = b_o.reshape(1, H)
    mol_vecs = _tc_readout(f_atoms, a3, wo1, wo2, bo, mol_ids_3d)
    return mol_vecs[:500]


# speedup vs baseline: 1.8467x; 1.0828x over previous
import functools

import jax
import jax.numpy as jnp
from jax import lax
from jax.experimental import pallas as pl
from jax.experimental.pallas import tpu as pltpu
from jax.experimental.pallas import tpu_sc as plsc

N = 10000
E = 320000
MAX_NB = 32
H = 128
NMOLS_PAD = 512

NC, NS = 2, 16
NW = NC * NS


def _rne_bf16_bits(b):
    return b + 0x7FFF + (lax.shift_right_arithmetic(b, 16) & 1)


def _pack_pair(lo_f32, hi_f32):
    blo = _rne_bf16_bits(lax.bitcast_convert_type(lo_f32, jnp.int32))
    bhi = _rne_bf16_bits(lax.bitcast_convert_type(hi_f32, jnp.int32))
    return ((lax.shift_right_arithmetic(blo, 16) & 0xFFFF)
            | (bhi & jnp.int32(-65536)))


def _mm_body(relu_in, pack_bf16, unpack_in, x_ref, w_ref, o_ref):
    w = w_ref[...]
    if unpack_in:
        xi = x_ref[...]
        lo = lax.bitcast_convert_type(lax.shift_left(xi, 16), jnp.float32)
        hi = lax.bitcast_convert_type(xi & jnp.int32(-65536), jnp.float32)
        if relu_in:
            lo = jnp.maximum(lo, 0.0)
            hi = jnp.maximum(hi, 0.0)
        k2 = w.shape[0] // 2
        y = (jnp.dot(lo, w[:k2], preferred_element_type=jnp.float32)
             + jnp.dot(hi, w[k2:], preferred_element_type=jnp.float32))
    else:
        x = x_ref[...]
        if relu_in:
            x = jnp.maximum(x, 0.0)
        y = jnp.dot(x, w, preferred_element_type=jnp.float32)
    if pack_bf16:
        o_ref[...] = _pack_cols(y)
    else:
        o_ref[...] = y


def _pack_cols(y):
    n = y.shape[1]
    b = lax.bitcast_convert_type(y, jnp.int32)
    lo = _rne_bf16_bits(b[:, : n // 2])
    hi = _rne_bf16_bits(b[:, n // 2:])
    return ((lax.shift_right_arithmetic(lo, 16) & 0xFFFF)
            | (hi & jnp.int32(-65536)))


def _mm_fused_body(x_ref, wi_ref, wh_ref, inp_ref, mw_ref):
    y1 = jnp.dot(x_ref[...], wi_ref[...], preferred_element_type=jnp.float32)
    inp_ref[...] = y1
    y2 = jnp.dot(jnp.maximum(y1, 0.0), wh_ref[...],
                 preferred_element_type=jnp.float32)
    mw_ref[...] = _pack_cols(y2)


def _tc_matmul_fused(x, wi, wh, block_rows=2000):
    m, k = x.shape
    n = wi.shape[1]
    grid = m // block_rows
    return pl.pallas_call(
        _mm_fused_body,
        grid=(grid,),
        in_specs=[
            pl.BlockSpec((block_rows, k), lambda i: (i, 0)),
            pl.BlockSpec((k, n), lambda i: (0, 0)),
            pl.BlockSpec((n, n), lambda i: (0, 0)),
        ],
        out_specs=[
            pl.BlockSpec((block_rows, n), lambda i: (i, 0)),
            pl.BlockSpec((block_rows, n // 2), lambda i: (i, 0)),
        ],
        out_shape=[
            jax.ShapeDtypeStruct((m, n), jnp.float32),
            jax.ShapeDtypeStruct((m, n // 2), jnp.int32),
        ],
        compiler_params=pltpu.CompilerParams(
            dimension_semantics=("parallel",)),
    )(x, wi, wh)


def _tc_matmul(x, w, relu_in, pack_bf16=False, unpack_in=False,
               block_rows=2000):
    m, k_in = x.shape
    k, n = w.shape
    n_out = n // 2 if pack_bf16 else n
    out_dtype = jnp.int32 if pack_bf16 else jnp.float32
    grid = m // block_rows
    return pl.pallas_call(
        functools.partial(_mm_body, relu_in, pack_bf16, unpack_in),
        grid=(grid,),
        in_specs=[
            pl.BlockSpec((block_rows, k_in), lambda i: (i, 0)),
            pl.BlockSpec((k, n), lambda i: (0, 0)),
        ],
        out_specs=pl.BlockSpec((block_rows, n_out), lambda i: (i, 0)),
        out_shape=jax.ShapeDtypeStruct((m, n_out), out_dtype),
        compiler_params=pltpu.CompilerParams(
            dimension_semantics=("parallel",)),
    )(x, w)


N_PAD = 10240
GS_ATOMS = N_PAD // NW
GS_BLOCKS = GS_ATOMS // 4


def _bf16_decode(w):
    lo = lax.bitcast_convert_type(lax.shift_left(w, 16), jnp.float32)
    hi = lax.bitcast_convert_type(
        w & jnp.full((16,), -65536, jnp.int32), jnp.float32)
    return lo, hi


def _make_gathersum(apply_relu, packed, pack_out=False):
    mesh = plsc.VectorSubcoreMesh(core_axis_name="c", subcore_axis_name="s")
    row_w = H // 2 if packed else H
    out_w = H // 2 if pack_out else H
    out_dt = jnp.int32 if pack_out else jnp.float32

    @functools.partial(
        pl.kernel,
        out_type=jax.ShapeDtypeStruct((N_PAD, out_w), out_dt),
        mesh=mesh,
        scratch_types=[
            pltpu.VMEM((GS_BLOCKS, 128), jnp.int32),
            pltpu.VMEM((128, row_w),
                       jnp.int32 if packed else jnp.float32),
            pltpu.VMEM((128, row_w),
                       jnp.int32 if packed else jnp.float32),
            pltpu.VMEM((GS_ATOMS, out_w), out_dt),
            pltpu.SemaphoreType.DMA,
            pltpu.SemaphoreType.DMA,
        ],
        compiler_params=pltpu.CompilerParams(use_tc_tiling_on_sc=False),
    )
    def gsum(mw_hbm, a2b_hbm, a_hbm, idx_v, rows0_v, rows1_v, aout_v,
             sem0, sem1):
        wid = lax.axis_index("s") * NC + lax.axis_index("c")
        base = wid * GS_ATOMS
        pltpu.sync_copy(a2b_hbm.at[pl.ds(wid * GS_BLOCKS, GS_BLOCKS)], idx_v)

        def issue(b, rows_v, sem):
            pltpu.async_copy(mw_hbm.at[idx_v.at[b]], rows_v, sem)

        def drain(rows_v, sem):
            pltpu.make_async_copy(mw_hbm.at[pl.ds(0, 128)], rows_v, sem).wait()

        def process_f32(b, rows_v):
            for j in range(4):
                for c in range(H // 16):
                    sl = pl.ds(c * 16, 16)
                    r0 = rows_v[j * MAX_NB, sl]
                    if apply_relu:
                        r0 = jnp.maximum(r0, 0.0)
                    acc = r0
                    for r in range(1, MAX_NB):
                        v = rows_v[j * MAX_NB + r, sl]
                        if apply_relu:
                            v = jnp.maximum(v, 0.0)
                        acc = acc + v
                    aout_v[b * 4 + j, sl] = acc

        def process_packed(b, rows_v):
            for j in range(4):
                for c in range(H // 32):
                    sl = pl.ds(c * 16, 16)
                    lo, hi = _bf16_decode(rows_v[j * MAX_NB, sl])
                    if apply_relu:
                        lo = jnp.maximum(lo, 0.0)
                        hi = jnp.maximum(hi, 0.0)
                    acc_lo, acc_hi = lo, hi
                    for r in range(1, MAX_NB):
                        lo, hi = _bf16_decode(rows_v[j * MAX_NB + r, sl])
                        if apply_relu:
                            lo = jnp.maximum(lo, 0.0)
                            hi = jnp.maximum(hi, 0.0)
                        acc_lo = acc_lo + lo
                        acc_hi = acc_hi + hi
                    if pack_out:
                        aout_v[b * 4 + j, pl.ds(c * 16, 16)] = _pack_pair(
                            acc_lo, acc_hi)
                    else:
                        aout_v[b * 4 + j, pl.ds(c * 16, 16)] = acc_lo
                        aout_v[b * 4 + j, pl.ds(H // 2 + c * 16, 16)] = acc_hi

        process = process_packed if packed else process_f32

        issue(0, rows0_v, sem0)

        def pair_body(i, _):
            b0 = 2 * i
            issue(b0 + 1, rows1_v, sem1)
            drain(rows0_v, sem0)
            process(b0, rows0_v)

            @pl.when(i < GS_BLOCKS // 2 - 1)
            def _():
                issue(b0 + 2, rows0_v, sem0)

            drain(rows1_v, sem1)
            process(b0 + 1, rows1_v)
            return 0

        lax.fori_loop(0, GS_BLOCKS // 2, pair_body, 0, unroll=False)
        pltpu.sync_copy(aout_v, a_hbm.at[pl.ds(base, GS_ATOMS)])

    return gsum


_gathersum_bf = _make_gathersum(False, packed=True, pack_out=True)
_gathersum_relu_bf = _make_gathersum(True, packed=True)


CB_STRIDE = 10000
CB_EDGES = 10240
CB_EB = 64
CB_BLOCKS = CB_EDGES // CB_EB


def _make_combine():
    mesh = plsc.VectorSubcoreMesh(core_axis_name="c", subcore_axis_name="s")

    @functools.partial(
        pl.kernel,
        out_type=jax.ShapeDtypeStruct((E, H // 2), jnp.int32),
        mesh=mesh,
        scratch_types=[
            pltpu.VMEM((CB_EDGES,), jnp.int32),
            pltpu.VMEM((CB_EDGES,), jnp.int32),
            pltpu.VMEM((CB_EB, H // 2), jnp.int32),
            pltpu.VMEM((CB_EB, H // 2), jnp.int32),
            pltpu.VMEM((CB_EB, H), jnp.float32),
            pltpu.VMEM((CB_EB, H // 2), jnp.int32),
            pltpu.VMEM((CB_EB, H // 2), jnp.int32),
            pltpu.VMEM((CB_EB, H // 2), jnp.int32),
            pltpu.VMEM((CB_EB, H), jnp.float32),
            pltpu.VMEM((CB_EB, H // 2), jnp.int32),
            pltpu.SemaphoreType.DMA, pltpu.SemaphoreType.DMA,
            pltpu.SemaphoreType.DMA, pltpu.SemaphoreType.DMA,
            pltpu.SemaphoreType.DMA, pltpu.SemaphoreType.DMA,
            pltpu.SemaphoreType.DMA, pltpu.SemaphoreType.DMA,
        ],
        compiler_params=pltpu.CompilerParams(use_tc_tiling_on_sc=False),
    )
    def combine(inp_hbm, a_hbm, mw_hbm, b2a_hbm, b2revb_hbm, out_hbm,
                aidx_v, ridx_v,
                ar0, mr0, ir0, or0, ar1, mr1, ir1, or1,
                sa0, sm0, si0, so0, sa1, sm1, si1, so1):
        wid = lax.axis_index("s") * NC + lax.axis_index("c")
        base = jnp.minimum(wid * CB_STRIDE, E - CB_EDGES)
        pltpu.sync_copy(b2a_hbm.at[pl.ds(base, CB_EDGES)], aidx_v)
        pltpu.sync_copy(b2revb_hbm.at[pl.ds(base, CB_EDGES)], ridx_v)

        def issue_in(b, ar, mr, ir, sa, sm, si):
            off = b * CB_EB
            pltpu.async_copy(a_hbm.at[aidx_v.at[pl.ds(off, CB_EB)]], ar, sa)
            pltpu.async_copy(mw_hbm.at[ridx_v.at[pl.ds(off, CB_EB)]], mr, sm)
            pltpu.async_copy(inp_hbm.at[pl.ds(base + off, CB_EB)], ir, si)

        def drain_in(ar, mr, ir, sa, sm, si):
            pltpu.make_async_copy(a_hbm.at[pl.ds(0, CB_EB)], ar, sa).wait()
            pltpu.make_async_copy(mw_hbm.at[pl.ds(0, CB_EB)], mr, sm).wait()
            pltpu.make_async_copy(inp_hbm.at[pl.ds(0, CB_EB)], ir, si).wait()

        def drain_out(orv, so):
            pltpu.make_async_copy(orv, out_hbm.at[pl.ds(0, CB_EB)], so).wait()

        def compute(ar, mr, ir, orv):
            def row_body(r, _):
                for c in range(H // 32):
                    slp = pl.ds(c * 16, 16)
                    mlo, mhi = _bf16_decode(mr[r, slp])
                    alo, ahi = _bf16_decode(ar[r, slp])
                    sl0 = pl.ds(c * 16, 16)
                    sl1 = pl.ds(H // 2 + c * 16, 16)
                    olo = ir[r, sl0] + alo - mlo
                    ohi = ir[r, sl1] + ahi - mhi
                    orv[r, slp] = _pack_pair(olo, ohi)
                return 0
            lax.fori_loop(0, CB_EB, row_body, 0, unroll=4)

        issue_in(0, ar0, mr0, ir0, sa0, sm0, si0)

        def pair_body(i, _):
            b0 = 2 * i
            issue_in(b0 + 1, ar1, mr1, ir1, sa1, sm1, si1)
            drain_in(ar0, mr0, ir0, sa0, sm0, si0)

            @pl.when(i > 0)
            def _():
                drain_out(or0, so0)

            compute(ar0, mr0, ir0, or0)
            pltpu.async_copy(or0, out_hbm.at[pl.ds(base + b0 * CB_EB, CB_EB)],
                             so0)

            @pl.when(i < CB_BLOCKS // 2 - 1)
            def _():
                issue_in(b0 + 2, ar0, mr0, ir0, sa0, sm0, si0)

            drain_in(ar1, mr1, ir1, sa1, sm1, si1)

            @pl.when(i > 0)
            def _():
                drain_out(or1, so1)

            compute(ar1, mr1, ir1, or1)
            pltpu.async_copy(
                or1, out_hbm.at[pl.ds(base + (b0 + 1) * CB_EB, CB_EB)], so1)
            return 0

        lax.fori_loop(0, CB_BLOCKS // 2, pair_body, 0, unroll=False)
        drain_out(or0, so0)
        drain_out(or1, so1)

    return combine


_combine = _make_combine()


RO_BLOCK = 2000
RO_GRID = N // RO_BLOCK


def _ro_body(fa_ref, a3_ref, wo1_ref, wo2_ref, bo_ref, mid_ref, out_ref,
             msum_ref, cnt_ref):
    i = pl.program_id(0)

    @pl.when(i == 0)
    def _():
        msum_ref[...] = jnp.zeros_like(msum_ref)
        cnt_ref[...] = jnp.zeros_like(cnt_ref)

    h = jnp.dot(fa_ref[...], wo1_ref[...], preferred_element_type=jnp.float32)
    h = h + jnp.dot(a3_ref[...], wo2_ref[...],
                    preferred_element_type=jnp.float32)
    h = jnp.maximum(h + bo_ref[...], 0.0)

    mid_row = mid_ref[0]
    mask_t = (lax.broadcasted_iota(jnp.int32, (NMOLS_PAD, RO_BLOCK), 0)
              == mid_row).astype(jnp.float32)
    msum_ref[...] += jnp.dot(mask_t, h, preferred_element_type=jnp.float32)
    cnt_ref[...] += jnp.broadcast_to(
        jnp.sum(mask_t, axis=1, keepdims=True), (NMOLS_PAD, H))

    @pl.when(i == RO_GRID - 1)
    def _():
        out_ref[...] = msum_ref[...] / jnp.maximum(cnt_ref[...], 1.0)


def _tc_readout(f_atoms, a3, wo1, wo2, bo, mol_ids_2d):
    return pl.pallas_call(
        _ro_body,
        grid=(RO_GRID,),
        in_specs=[
            pl.BlockSpec((RO_BLOCK, H), lambda i: (i, 0)),
            pl.BlockSpec((RO_BLOCK, H), lambda i: (i, 0)),
            pl.BlockSpec((H, H), lambda i: (0, 0)),
            pl.BlockSpec((H, H), lambda i: (0, 0)),
            pl.BlockSpec((1, H), lambda i: (0, 0)),
            pl.BlockSpec((1, 1, RO_BLOCK), lambda i: (i, 0, 0)),
        ],
        out_specs=pl.BlockSpec((NMOLS_PAD, H), lambda i: (0, 0)),
        out_shape=jax.ShapeDtypeStruct((NMOLS_PAD, H), jnp.float32),
        scratch_shapes=[
            pltpu.VMEM((NMOLS_PAD, H), jnp.float32),
            pltpu.VMEM((NMOLS_PAD, H), jnp.float32),
        ],
        compiler_params=pltpu.CompilerParams(
            dimension_semantics=("arbitrary",)),
    )(f_atoms, a3, wo1, wo2, bo, mol_ids_2d)


def kernel(f_atoms, f_bonds, a2b, b2a, b2revb, mol_ids, W_i, W_h, W_o, b_o):
    a2b = a2b.astype(jnp.int32)
    b2a = b2a.astype(jnp.int32)
    b2revb = b2revb.astype(jnp.int32)
    a2b_pad = jnp.zeros((N_PAD, MAX_NB), jnp.int32).at[:N].set(a2b)
    a2b_r = a2b_pad.reshape(N_PAD * MAX_NB // 128, 128)

    inp, mw_i = _tc_matmul_fused(f_bonds, W_i, W_h)
    for it in range(2):
        if it > 0:
            mw_i = _tc_matmul(pre, W_h, relu_in=True,
                              pack_bf16=True,
                              unpack_in=True)
        a_sum = _gathersum_bf(mw_i, a2b_r)
        pre = _combine(inp, a_sum, mw_i, b2a, b2revb)

    a3 = _gathersum_relu_bf(pre, a2b_r)[:N]

    mol_ids_3d = mol_ids.astype(jnp.int32).reshape(RO_GRID, 1, RO_BLOCK)
    wo1 = W_o[:H]
    wo2 = W_o[H:]
    bo = b_o.reshape(1, H)
    mol_vecs = _tc_readout(f_atoms, a3, wo1, wo2, bo, mol_ids_3d)
    return mol_vecs[:500]
